# baseline (speedup 1.0000x reference)
import jax
import jax.numpy as jnp
from jax import lax
from jax.experimental import pallas as pl
from jax.experimental.pallas import tpu as pltpu
from jax.experimental.pallas import tpu_sc as plsc

_NC = 2
_NS = 16
_NW = _NC * _NS

_BATCH = 16384
_EMB_DIM = 16
_B_PER_W = _BATCH // _NW


def _gather_body(y_hbm, table_hbm, out_hbm, idx_v, rows_v, sem):
    wid = lax.axis_index("s") * _NC + lax.axis_index("c")
    base = wid * _B_PER_W
    pltpu.sync_copy(y_hbm.at[pl.ds(base, _B_PER_W)], idx_v)

    def issue(g, _):
        vec = idx_v[pl.ds(g * 16, 16)]
        for lane in range(16):
            pltpu.make_async_copy(
                table_hbm.at[pl.ds(vec[lane], 1)],
                rows_v.at[pl.ds(g * 16 + lane, 1)],
                sem.at[lane % 4],
            ).start()
        return ()

    lax.fori_loop(0, _B_PER_W // 16, issue, ())
    for q in range(4):
        pltpu.make_async_copy(
            table_hbm.at[pl.ds(0, _B_PER_W // 4)],
            rows_v.at[pl.ds(0, _B_PER_W // 4)],
            sem.at[q],
        ).wait()
    pltpu.sync_copy(rows_v, out_hbm.at[pl.ds(base, _B_PER_W)])


@jax.jit
def _gather(y, emb_table):
    mesh = plsc.VectorSubcoreMesh(core_axis_name="c", subcore_axis_name="s")
    kern = pl.kernel(
        _gather_body,
        out_type=jax.ShapeDtypeStruct((_BATCH, _EMB_DIM), jnp.float32),
        mesh=mesh,
        scratch_types=[
            pltpu.VMEM((_B_PER_W,), jnp.int32),
            pltpu.VMEM((_B_PER_W, _EMB_DIM), jnp.float32),
            pltpu.SemaphoreType.DMA((4,)),
        ],
    )
    return kern(y, emb_table)


def kernel(y, emb_table):
    return _gather(y.astype(jnp.int32), emb_table)
